# baseline (device time: 16517 ns/iter reference)
import jax
import jax.numpy as jnp
from jax import lax
from jax.experimental import pallas as pl
from jax.experimental.pallas import tpu as pltpu

N_DEV = 4
N_TOK = 512
D_MODEL = 256
D_OUT = 512
N_EXP = 8
ROWS_PER = N_TOK // N_DEV

_SEND_ORDER = (2, 1, 3)


def kernel(x, router_W, route_idx, expert_W, shared_W):
    me = lax.axis_index("i")
    scores = x @ router_W
    scores = scores - jnp.max(scores, axis=1, keepdims=True)
    e = jnp.exp(scores)
    probs = e / jnp.sum(e, axis=1, keepdims=True)
    p = jnp.take_along_axis(probs, route_idx, axis=1)
    scales = jnp.concatenate(
        [
            p * (route_idx == 2 * me).astype(jnp.float32),
            p * (route_idx == 2 * me + 1).astype(jnp.float32),
        ],
        axis=1,
    )

    def body(
        x_hbm,
        scales_hbm,
        expert_hbm,
        shared_hbm,
        out_hbm,
        x_ref,
        scale_ref,
        expert_ref,
        shared_ref,
        partial_ref,
        comm_ref,
        acc_ref,
        in_sems,
        out_sem,
        send_sems,
        recv_sems,
    ):
        my = lax.axis_index("i")

        barrier_sem = pltpu.get_barrier_semaphore()
        for h in range(1, N_DEV):
            peer = lax.rem(my + h, N_DEV)
            pl.semaphore_signal(
                barrier_sem,
                inc=1,
                device_id=(peer,),
                device_id_type=pl.DeviceIdType.MESH,
            )

        w_dma = pltpu.make_async_copy(expert_hbm, expert_ref, in_sems.at[0])
        x_dma = pltpu.make_async_copy(x_hbm, x_ref, in_sems.at[1])
        c_dma = pltpu.make_async_copy(scales_hbm, scale_ref, in_sems.at[2])
        s_dma = pltpu.make_async_copy(shared_hbm, shared_ref, in_sems.at[3])
        w_dma.start()
        x_dma.start()
        c_dma.start()
        s_dma.start()

        w_dma.wait()
        w0 = expert_ref[0].astype(jnp.bfloat16)
        w1 = expert_ref[1].astype(jnp.bfloat16)
        x_dma.wait()
        c_dma.wait()

        pl.semaphore_wait(barrier_sem, N_DEV - 1)

        sends = []
        for h in _SEND_ORDER:
            tgt = lax.rem(my + h, N_DEV)
            row0 = tgt * ROWS_PER
            xb = x_ref[pl.ds(row0, ROWS_PER), :].astype(jnp.bfloat16)
            y0 = jnp.dot(xb, w0, preferred_element_type=jnp.float32)
            y1 = jnp.dot(xb, w1, preferred_element_type=jnp.float32)
            s0 = scale_ref[pl.ds(row0, ROWS_PER), 0:1]
            s1 = scale_ref[pl.ds(row0, ROWS_PER), 1:2]
            partial_ref[h - 1] = (s0 * y0 + s1 * y1).astype(jnp.bfloat16)
            rdma = pltpu.make_async_remote_copy(
                src_ref=partial_ref.at[h - 1],
                dst_ref=comm_ref.at[h - 1],
                send_sem=send_sems.at[h - 1],
                recv_sem=recv_sems.at[h - 1],
                device_id=(tgt,),
                device_id_type=pl.DeviceIdType.MESH,
            )
            rdma.start()
            sends.append(rdma)

        row0 = my * ROWS_PER
        xb = x_ref[pl.ds(row0, ROWS_PER), :].astype(jnp.bfloat16)
        y0 = jnp.dot(xb, w0, preferred_element_type=jnp.float32)
        y1 = jnp.dot(xb, w1, preferred_element_type=jnp.float32)
        s_dma.wait()
        shared = jnp.dot(
            xb, shared_ref[:, :].astype(jnp.bfloat16), preferred_element_type=jnp.float32
        )
        s0 = scale_ref[pl.ds(row0, ROWS_PER), 0:1]
        s1 = scale_ref[pl.ds(row0, ROWS_PER), 1:2]
        acc = shared + s0 * y0 + s1 * y1

        for h, rdma in zip(_SEND_ORDER, sends):
            rdma.wait()
            acc = acc + comm_ref[h - 1].astype(jnp.float32)

        acc_ref[:, :] = acc
        out_dma = pltpu.make_async_copy(acc_ref, out_hbm, out_sem)
        out_dma.start()
        out_dma.wait()

    return pl.pallas_call(
        body,
        out_shape=jax.ShapeDtypeStruct((ROWS_PER, D_OUT), jnp.float32),
        in_specs=[
            pl.BlockSpec(memory_space=pl.ANY),
            pl.BlockSpec(memory_space=pl.ANY),
            pl.BlockSpec(memory_space=pl.ANY),
            pl.BlockSpec(memory_space=pl.ANY),
        ],
        out_specs=pl.BlockSpec(memory_space=pltpu.MemorySpace.HBM),
        scratch_shapes=[
            pltpu.VMEM((N_TOK, D_MODEL), jnp.float32),
            pltpu.VMEM((N_TOK, 2), jnp.float32),
            pltpu.VMEM((2, D_MODEL, D_OUT), jnp.float32),
            pltpu.VMEM((D_MODEL, D_OUT), jnp.float32),
            pltpu.VMEM((N_DEV - 1, ROWS_PER, D_OUT), jnp.bfloat16),
            pltpu.VMEM((N_DEV - 1, ROWS_PER, D_OUT), jnp.bfloat16),
            pltpu.VMEM((ROWS_PER, D_OUT), jnp.float32),
            pltpu.SemaphoreType.DMA((4,)),
            pltpu.SemaphoreType.DMA,
            pltpu.SemaphoreType.DMA((N_DEV - 1,)),
            pltpu.SemaphoreType.DMA((N_DEV - 1,)),
        ],
        compiler_params=pltpu.CompilerParams(collective_id=0),
    )(
        pltpu.with_memory_space_constraint(x, pltpu.MemorySpace.HBM),
        pltpu.with_memory_space_constraint(scales, pltpu.MemorySpace.HBM),
        pltpu.with_memory_space_constraint(expert_W, pltpu.MemorySpace.HBM),
        pltpu.with_memory_space_constraint(shared_W, pltpu.MemorySpace.HBM),
    )


# device time: 11037 ns/iter; 1.4965x vs baseline; 1.4965x over previous
import jax
import jax.numpy as jnp
from jax import lax
from jax.experimental import pallas as pl
from jax.experimental.pallas import tpu as pltpu

N_DEV = 4
N_TOK = 512
D_MODEL = 256
D_OUT = 512
N_EXP = 8
ROWS_PER = N_TOK // N_DEV

_SEND_ORDER = (2, 1, 3)


def kernel(x, router_W, route_idx, expert_W, shared_W):
    def body(
        x_hbm,
        router_ref,
        ridx_ref,
        expert_hbm,
        shared_hbm,
        out_hbm,
        x_ref,
        expert_ref,
        shared_ref,
        scale_ref,
        partial_ref,
        comm_ref,
        acc_ref,
        in_sems,
        out_sem,
        send_sems,
        recv_sems,
    ):
        me = lax.axis_index("i")

        barrier_sem = pltpu.get_barrier_semaphore()
        for h in range(1, N_DEV):
            peer = lax.rem(me + h, N_DEV)
            pl.semaphore_signal(
                barrier_sem,
                inc=1,
                device_id=(peer,),
                device_id_type=pl.DeviceIdType.MESH,
            )

        x_dma = pltpu.make_async_copy(x_hbm, x_ref, in_sems.at[0])
        w_dma = pltpu.make_async_copy(expert_hbm, expert_ref, in_sems.at[1])
        s_dma = pltpu.make_async_copy(shared_hbm, shared_ref, in_sems.at[2])
        x_dma.start()
        w_dma.start()
        s_dma.start()

        x_dma.wait()
        xv = x_ref[:, :]
        scores = lax.dot_general(
            xv,
            router_ref[:, :],
            dimension_numbers=(((1,), (1,)), ((), ())),
            preferred_element_type=jnp.float32,
        )
        scores = scores - jnp.max(scores, axis=1, keepdims=True)
        e = jnp.exp(scores)
        probs = e / jnp.sum(e, axis=1, keepdims=True)
        ridx = ridx_ref[:, :]
        col = lax.broadcasted_iota(jnp.int32, (N_TOK, N_EXP), 1)
        p = jnp.sum(jnp.where(col == ridx, probs, 0.0), axis=1, keepdims=True)
        scale_ref[:, 0:1] = p * (ridx == 2 * me).astype(jnp.float32)
        scale_ref[:, 1:2] = p * (ridx == 2 * me + 1).astype(jnp.float32)

        w_dma.wait()
        w0 = expert_ref[0].astype(jnp.bfloat16)
        w1 = expert_ref[1].astype(jnp.bfloat16)

        pl.semaphore_wait(barrier_sem, N_DEV - 1)

        sends = []
        for h in _SEND_ORDER:
            tgt = lax.rem(me + h, N_DEV)
            row0 = tgt * ROWS_PER
            xb = x_ref[pl.ds(row0, ROWS_PER), :].astype(jnp.bfloat16)
            y0 = jnp.dot(xb, w0, preferred_element_type=jnp.float32)
            y1 = jnp.dot(xb, w1, preferred_element_type=jnp.float32)
            s0 = scale_ref[pl.ds(row0, ROWS_PER), 0:1]
            s1 = scale_ref[pl.ds(row0, ROWS_PER), 1:2]
            partial_ref[h - 1] = (s0 * y0 + s1 * y1).astype(jnp.bfloat16)
            rdma = pltpu.make_async_remote_copy(
                src_ref=partial_ref.at[h - 1],
                dst_ref=comm_ref.at[h - 1],
                send_sem=send_sems.at[h - 1],
                recv_sem=recv_sems.at[h - 1],
                device_id=(tgt,),
                device_id_type=pl.DeviceIdType.MESH,
            )
            rdma.start()
            sends.append(rdma)

        row0 = me * ROWS_PER
        xb = x_ref[pl.ds(row0, ROWS_PER), :].astype(jnp.bfloat16)
        y0 = jnp.dot(xb, w0, preferred_element_type=jnp.float32)
        y1 = jnp.dot(xb, w1, preferred_element_type=jnp.float32)
        s_dma.wait()
        shared = jnp.dot(
            xb, shared_ref[:, :].astype(jnp.bfloat16), preferred_element_type=jnp.float32
        )
        s0 = scale_ref[pl.ds(row0, ROWS_PER), 0:1]
        s1 = scale_ref[pl.ds(row0, ROWS_PER), 1:2]
        acc = shared + s0 * y0 + s1 * y1

        for h, rdma in zip(_SEND_ORDER, sends):
            rdma.wait()
            acc = acc + comm_ref[h - 1].astype(jnp.float32)

        acc_ref[:, :] = acc
        out_dma = pltpu.make_async_copy(acc_ref, out_hbm, out_sem)
        out_dma.start()
        out_dma.wait()

    return pl.pallas_call(
        body,
        out_shape=jax.ShapeDtypeStruct((ROWS_PER, D_OUT), jnp.float32),
        in_specs=[
            pl.BlockSpec(memory_space=pl.ANY),
            pl.BlockSpec(memory_space=pltpu.VMEM),
            pl.BlockSpec(memory_space=pltpu.VMEM),
            pl.BlockSpec(memory_space=pl.ANY),
            pl.BlockSpec(memory_space=pl.ANY),
        ],
        out_specs=pl.BlockSpec(memory_space=pltpu.MemorySpace.HBM),
        scratch_shapes=[
            pltpu.VMEM((N_TOK, D_MODEL), jnp.float32),
            pltpu.VMEM((2, D_MODEL, D_OUT), jnp.float32),
            pltpu.VMEM((D_MODEL, D_OUT), jnp.float32),
            pltpu.VMEM((N_TOK, 2), jnp.float32),
            pltpu.VMEM((N_DEV - 1, ROWS_PER, D_OUT), jnp.bfloat16),
            pltpu.VMEM((N_DEV - 1, ROWS_PER, D_OUT), jnp.bfloat16),
            pltpu.VMEM((ROWS_PER, D_OUT), jnp.float32),
            pltpu.SemaphoreType.DMA((3,)),
            pltpu.SemaphoreType.DMA,
            pltpu.SemaphoreType.DMA((N_DEV - 1,)),
            pltpu.SemaphoreType.DMA((N_DEV - 1,)),
        ],
        compiler_params=pltpu.CompilerParams(collective_id=0),
    )(
        pltpu.with_memory_space_constraint(x, pltpu.MemorySpace.HBM),
        router_W.T,
        route_idx,
        pltpu.with_memory_space_constraint(expert_W, pltpu.MemorySpace.HBM),
        pltpu.with_memory_space_constraint(shared_W, pltpu.MemorySpace.HBM),
    )
